# Initial kernel scaffold; baseline (speedup 1.0000x reference)
import jax
import jax.numpy as jnp
from jax.experimental import pallas as pl


def kernel(logits, labels):
    raise NotImplementedError("write your pallas kernel here")


# speedup vs baseline: 2.0244x; 2.0244x over previous
import functools

import jax
import jax.numpy as jnp
from jax import lax
from jax.experimental import pallas as pl
from jax.experimental.pallas import tpu as pltpu
from jax.experimental.pallas import tpu_sc as plsc

N_TOTAL = 2097152
N_BINS_OUT = 10
NC = 2
NS = 16
L = 16
NW = NC * NS
PER_W = N_TOTAL // NW
BLK = 8192
NBLK = PER_W // BLK
HBINS = 11
HWORDS = HBINS * L

_mesh = plsc.VectorSubcoreMesh(core_axis_name="c", subcore_axis_name="s")


@functools.partial(
    pl.kernel,
    mesh=_mesh,
    out_type=(
        jax.ShapeDtypeStruct((NW, HWORDS), jnp.int32),
        jax.ShapeDtypeStruct((NW, HWORDS), jnp.int32),
        jax.ShapeDtypeStruct((NW, HWORDS), jnp.float32),
    ),
    scratch_types=[
        pltpu.VMEM((2 * BLK,), jnp.float32),
        pltpu.VMEM((BLK,), jnp.int32),
        pltpu.VMEM((HWORDS,), jnp.int32),
        pltpu.VMEM((HWORDS,), jnp.int32),
        pltpu.VMEM((HWORDS,), jnp.float32),
    ],
    compiler_params=pltpu.CompilerParams(needs_layout_passes=False),
)
def _ece_hist(lg_hbm, lb_hbm, cnt_out, lab_out, prd_out,
              lg_v, lb_v, cnt_v, lab_v, prd_v):
    wid = lax.axis_index("s") * NC + lax.axis_index("c")

    lane = lax.iota(jnp.int32, L)
    two_iota = lane * 2
    ones_i = jnp.ones((L,), jnp.int32)
    z_i = jnp.zeros((L,), jnp.int32)
    z_f = jnp.zeros((L,), jnp.float32)

    for b in range(HBINS):
        cnt_v[pl.ds(b * L, L)] = z_i
        lab_v[pl.ds(b * L, L)] = z_i
        prd_v[pl.ds(b * L, L)] = z_f

    elem0 = wid * PER_W

    def do_block(blk):
        off = elem0 + blk * BLK
        pltpu.sync_copy(lg_hbm.at[pl.ds(off * 2, 2 * BLK)], lg_v)
        pltpu.sync_copy(lb_hbm.at[pl.ds(off, BLK)], lb_v)

        def body(j, carry):
            base = j * (2 * L)
            idx0 = two_iota + base
            idx1 = idx0 + 1
            l0 = plsc.load_gather(lg_v, [idx0])
            l1 = plsc.load_gather(lg_v, [idx1])
            lb16 = lb_v[pl.ds(j * L, L)]
            e = jnp.exp(l0 - l1)
            p = 1.0 / (1.0 + e)
            bin_ = (p * 10.0).astype(jnp.int32)
            addr = bin_ * L + lane
            plsc.addupdate_scatter(cnt_v, [addr], ones_i)
            plsc.addupdate_scatter(lab_v, [addr], lb16)
            plsc.addupdate_scatter(prd_v, [addr], p)
            return carry

        lax.fori_loop(0, BLK // L, body, 0)

    for blk in range(NBLK):
        do_block(blk)

    pltpu.sync_copy(cnt_v, cnt_out.at[wid])
    pltpu.sync_copy(lab_v, lab_out.at[wid])
    pltpu.sync_copy(prd_v, prd_out.at[wid])


def kernel(logits, labels):
    cnt, lab, prd = _ece_hist(logits.reshape(-1), labels)
    sizes = cnt.reshape(NW, HBINS, L).sum(axis=(0, 2))[:N_BINS_OUT]
    lab_s = lab.reshape(NW, HBINS, L).sum(axis=(0, 2))[:N_BINS_OUT]
    prd_s = prd.reshape(NW, HBINS, L).sum(axis=(0, 2))[:N_BINS_OUT]
    sizes = sizes.astype(jnp.float32)
    lab_s = lab_s.astype(jnp.float32)
    nonempty = sizes > 0
    safe = jnp.where(nonempty, sizes, 1.0)
    accs = jnp.where(nonempty, lab_s / safe, 0.0)
    confs = jnp.where(nonempty, prd_s / safe, 0.0)
    return jnp.sum(sizes / jnp.sum(sizes) * jnp.abs(accs - confs))
